# baseline (speedup 1.0000x reference)
import functools

import jax
import jax.numpy as jnp
from jax import lax
from jax.experimental import pallas as pl
from jax.experimental.pallas import tpu as pltpu
from jax.experimental.pallas import tpu_sc as plsc

LATENT = 64
GCHUNK = 128


@functools.cache
def _build(B: int):
    info = plsc.get_sparse_core_info()
    NC, NS, L = info.num_cores, info.num_subcores, info.num_lanes
    NW = NC * NS
    bpw = B // NW
    nchunks = bpw // GCHUNK
    ngroups = bpw // L
    idx_rows = B // GCHUNK
    rows_per_w = bpw // GCHUNK

    mesh = plsc.VectorSubcoreMesh(core_axis_name="c", subcore_axis_name="s")

    @functools.partial(
        pl.kernel,
        mesh=mesh,
        out_type=jax.ShapeDtypeStruct((B,), jnp.float32),
        scratch_types=[
            pltpu.VMEM((rows_per_w, GCHUNK), jnp.int32),
            pltpu.VMEM((rows_per_w, GCHUNK), jnp.int32),
            pltpu.VMEM((bpw, LATENT), jnp.float32),
            pltpu.VMEM((bpw, LATENT), jnp.float32),
            pltpu.VMEM((LATENT,), jnp.float32),
            pltpu.VMEM((bpw,), jnp.float32),
            pltpu.SemaphoreType.DMA,
            pltpu.SemaphoreType.DMA,
        ],
        compiler_params=pltpu.CompilerParams(
            needs_layout_passes=False, use_tc_tiling_on_sc=False
        ),
    )
    def gmf(uidx_hbm, iidx_hbm, utab_hbm, itab_hbm, w_hbm, out_hbm,
            uidx_v, iidx_v, urows_v, irows_v, w_v, out_v, sem_u, sem_i):
        wid = lax.axis_index("s") * NC + lax.axis_index("c")
        base = wid * bpw
        row0 = wid * rows_per_w

        pltpu.sync_copy(uidx_hbm.at[pl.ds(row0, rows_per_w)], uidx_v)
        pltpu.sync_copy(iidx_hbm.at[pl.ds(row0, rows_per_w)], iidx_v)
        pltpu.sync_copy(w_hbm, w_v)

        copies = []
        for c in range(nchunks):
            dst = urows_v.at[pl.ds(c * GCHUNK, GCHUNK)]
            copies.append(pltpu.async_copy(utab_hbm.at[uidx_v.at[c]], dst, sem_u))
            dst = irows_v.at[pl.ds(c * GCHUNK, GCHUNK)]
            copies.append(pltpu.async_copy(itab_hbm.at[iidx_v.at[c]], dst, sem_i))
        for cp in copies:
            cp.wait()

        lanes = lax.iota(jnp.int32, L)
        w_chunks = [w_v[pl.ds(k * L, L)] for k in range(LATENT // L)]

        def group_body(g, _):
            row_idx = g * L + lanes
            acc = jnp.zeros((L,), jnp.float32)
            for d in range(LATENT):
                col = jnp.full((L,), d, jnp.int32)
                u = plsc.load_gather(urows_v, [row_idx, col])
                v = plsc.load_gather(irows_v, [row_idx, col])
                acc = acc + (u * v) * w_chunks[d // L][d % L]
            p = 1.0 / (1.0 + jnp.exp(-acc))
            out_v[pl.ds(g * L, L)] = p
            return _

        lax.fori_loop(0, ngroups, group_body, None)

        pltpu.sync_copy(out_v, out_hbm.at[pl.ds(base, bpw)])

    return gmf, idx_rows


def kernel(user_indices, item_indices, user_weight, item_weight, linear_weight):
    B = user_indices.shape[0]
    gmf, idx_rows = _build(B)
    uidx = user_indices.astype(jnp.int32).reshape(idx_rows, GCHUNK)
    iidx = item_indices.astype(jnp.int32).reshape(idx_rows, GCHUNK)
    w = linear_weight.reshape(LATENT).astype(jnp.float32)
    out = gmf(uidx, iidx, user_weight, item_weight, w)
    return out.reshape(B, 1)
